# baseline (speedup 1.0000x reference)
import functools

import jax
import jax.numpy as jnp
from jax import lax
from jax.experimental import pallas as pl
from jax.experimental.pallas import tpu as pltpu
from jax.experimental.pallas import tpu_sc as plsc

NUM_CLASSES = 1000
ROWS = 4096 * 20
NC, NS, L = 2, 16, 16
NW = NC * NS
RPW = ROWS // NW
BATCH = 64
NBUF = 2
NBATCH = RPW // BATCH
GROUPS = BATCH // L
BUF_WORDS = BATCH * NUM_CLASSES


def _body(x_hbm, out_hbm, idx_v, buf0, buf1, sem0, sem1):
  wid = lax.axis_index("s") * NC + lax.axis_index("c")
  base = wid * RPW

  pltpu.sync_copy(x_hbm.at[pl.ds(base * 1, RPW)], idx_v)

  zeros = jnp.zeros((L,), jnp.int32)
  ones = jnp.ones((L,), jnp.int32)
  bufs = (buf0, buf1)
  sems = (sem0, sem1)

  def _zero(i, _):
    buf0[pl.ds(i * L, L)] = zeros
    buf1[pl.ds(i * L, L)] = zeros
    return 0

  lax.fori_loop(0, BUF_WORDS // L, _zero, 0)

  iota = lax.iota(jnp.int32, L)
  rowoff = [(iota + g * L) * NUM_CLASSES for g in range(GROUPS)]

  def _step(i, _):
    for b in range(NBUF):
      tt = i * NBUF + b
      buf = bufs[b]
      sem = sems[b]
      offs = (base + tt * BATCH) * NUM_CLASSES

      @pl.when(i >= 1)
      def _drain():
        pltpu.make_async_copy(
            buf, out_hbm.at[pl.ds(offs, BUF_WORDS)], sem).wait()
        prev = (tt - NBUF) * BATCH
        for g in range(GROUPS):
          col = idx_v[pl.ds(prev + g * L, L)]
          plsc.store_scatter(buf, [rowoff[g] + col], zeros)

      cur = tt * BATCH
      for g in range(GROUPS):
        col = idx_v[pl.ds(cur + g * L, L)]
        plsc.store_scatter(buf, [rowoff[g] + col], ones)
      pltpu.async_copy(buf, out_hbm.at[pl.ds(offs, BUF_WORDS)], sem)
    return 0

  lax.fori_loop(0, NBATCH // NBUF, _step, 0)

  for b in range(NBUF):
    tt = NBATCH - NBUF + b
    offs = (base + tt * BATCH) * NUM_CLASSES
    pltpu.make_async_copy(
        bufs[b], out_hbm.at[pl.ds(offs, BUF_WORDS)], sems[b]).wait()


@jax.jit
def _one_hot_sc(xf):
  mesh = plsc.VectorSubcoreMesh(core_axis_name="c", subcore_axis_name="s")
  k = pl.kernel(
      _body,
      out_type=jax.ShapeDtypeStruct((ROWS * NUM_CLASSES,), jnp.int32),
      mesh=mesh,
      scratch_types=[
          pltpu.VMEM((RPW,), jnp.int32),
          pltpu.VMEM((BUF_WORDS,), jnp.int32),
          pltpu.VMEM((BUF_WORDS,), jnp.int32),
          pltpu.SemaphoreType.DMA,
          pltpu.SemaphoreType.DMA,
      ],
      compiler_params=pltpu.CompilerParams(needs_layout_passes=False),
  )
  return k(xf)


def kernel(x):
  xf = x.reshape(-1).astype(jnp.int32)
  out = _one_hot_sc(xf)
  return out.reshape(4096, 20, NUM_CLASSES)


# speedup vs baseline: 1.4862x; 1.4862x over previous
import jax
import jax.numpy as jnp
from jax import lax
from jax.experimental import pallas as pl
from jax.experimental.pallas import tpu as pltpu
from jax.experimental.pallas import tpu_sc as plsc

NUM_CLASSES = 1000
B, S = 4096, 20
NC, NS, L = 2, 16, 16
NW = NC * NS
BPW = B // NW
NB = 2
ROWS_PER_BLK = NB * S
NBUF = 2
NBLK = BPW // NB
RPW = BPW * S


def _body(x_hbm, out_hbm, idx_v, buf0, buf1, sem0, sem1):
  wid = lax.axis_index("s") * NC + lax.axis_index("c")
  b_base = wid * BPW

  pltpu.sync_copy(x_hbm.at[pl.ds(wid * RPW, RPW)], idx_v)

  zeros = jnp.zeros((L,), jnp.int32)
  ones = jnp.ones((L,), jnp.int32)
  bufs = (buf0, buf1)
  sems = (sem0, sem1)

  def _zero(r, _):
    i = r // S
    j = r % S
    for k in list(range(0, NUM_CLASSES - L, L)) + [NUM_CLASSES - L]:
      buf0[i, j, pl.ds(k, L)] = zeros
      buf1[i, j, pl.ds(k, L)] = zeros
    return 0

  lax.fori_loop(0, ROWS_PER_BLK, _zero, 0)

  iota = lax.iota(jnp.int32, L)
  groups = []
  r0 = 0
  while r0 < ROWS_PER_BLK:
    g_rows = min(L, ROWS_PER_BLK - r0)
    rr = jnp.minimum(iota, g_rows - 1) + r0
    iv = rr // S
    jv = rr - iv * S
    mask = (iota < g_rows) if g_rows < L else None
    groups.append((r0, iv, jv, mask))
    r0 += g_rows

  def _scatter(buf, row_start, val):
    for (g_off, iv, jv, mask) in groups:
      col = idx_v[pl.ds(row_start + g_off, L)]
      plsc.store_scatter(buf, [iv, jv, col], val, mask=mask)

  def _step(i, _):
    for b in range(NBUF):
      tt = i * NBUF + b
      buf = bufs[b]
      sem = sems[b]
      dst = out_hbm.at[pl.ds(b_base + tt * NB, NB)]

      @pl.when(i >= 1)
      def _drain():
        pltpu.make_async_copy(buf, dst, sem).wait()
        _scatter(buf, (tt - NBUF) * ROWS_PER_BLK, zeros)

      _scatter(buf, tt * ROWS_PER_BLK, ones)
      pltpu.async_copy(buf, dst, sem)
    return 0

  lax.fori_loop(0, NBLK // NBUF, _step, 0)

  for b in range(NBUF):
    tt = NBLK - NBUF + b
    dst = out_hbm.at[pl.ds(b_base + tt * NB, NB)]
    pltpu.make_async_copy(bufs[b], dst, sems[b]).wait()


@jax.jit
def _one_hot_sc(xf):
  mesh = plsc.VectorSubcoreMesh(core_axis_name="c", subcore_axis_name="s")
  k = pl.kernel(
      _body,
      out_type=jax.ShapeDtypeStruct((B, S, NUM_CLASSES), jnp.int32),
      mesh=mesh,
      scratch_types=[
          pltpu.VMEM((RPW,), jnp.int32),
          pltpu.VMEM((NB, S, NUM_CLASSES), jnp.int32),
          pltpu.VMEM((NB, S, NUM_CLASSES), jnp.int32),
          pltpu.SemaphoreType.DMA,
          pltpu.SemaphoreType.DMA,
      ],
      compiler_params=pltpu.CompilerParams(
          needs_layout_passes=False,
          use_tc_tiling_on_sc=True,
      ),
  )
  return k(xf)


def kernel(x):
  xf = x.reshape(-1).astype(jnp.int32)
  return _one_hot_sc(xf)


# speedup vs baseline: 5.8135x; 3.9116x over previous
import jax
import jax.numpy as jnp
from jax import lax
from jax.experimental import pallas as pl
from jax.experimental.pallas import tpu as pltpu
from jax.experimental.pallas import tpu_sc as plsc

NUM_CLASSES = 1000
B, S = 4096, 20
NC, NS, L = 2, 16, 16
NW = NC * NS
BPW = B // NW
GPW = BPW // L
CH = 200
NCH = NUM_CLASSES // CH
NBLK = S * NCH
NBUF = 2


def _body(xt_hbm, out_hbm, idx_v, buf0, buf1, sem0, sem1):
  wid = lax.axis_index("s") * NC + lax.axis_index("c")
  b0 = wid * BPW

  pltpu.sync_copy(xt_hbm.at[:, pl.ds(b0, BPW)], idx_v)

  zeros = jnp.zeros((L,), jnp.int32)
  ones = jnp.ones((L,), jnp.int32)
  bufs = (buf0, buf1)
  sems = (sem0, sem1)

  def _zero(c, _):
    for g in range(GPW):
      buf0[c, pl.ds(g * L, L)] = zeros
      buf1[c, pl.ds(g * L, L)] = zeros
    return 0

  lax.fori_loop(0, CH, _zero, 0)

  iota = lax.iota(jnp.int32, L)

  def _scatter(buf, j, c0, val):
    for g in range(GPW):
      cvec = idx_v[j, pl.ds(g * L, L)]
      mask = (cvec >= c0) & (cvec < c0 + CH)
      plsc.store_scatter(buf, [cvec - c0, iota + g * L], val, mask=mask)

  def _step(i, _):
    for b in range(NBUF):
      tt = i * NBUF + b
      buf = bufs[b]
      sem = sems[b]
      j = tt // NCH
      c0 = (tt - j * NCH) * CH
      dst = out_hbm.at[j, pl.ds(c0, CH), pl.ds(b0, BPW)]

      @pl.when(i >= 1)
      def _drain():
        pltpu.make_async_copy(buf, dst, sem).wait()
        tp = tt - NBUF
        jp = tp // NCH
        _scatter(buf, jp, (tp - jp * NCH) * CH, zeros)

      _scatter(buf, j, c0, ones)
      pltpu.async_copy(buf, dst, sem)
    return 0

  lax.fori_loop(0, NBLK // NBUF, _step, 0)

  for b in range(NBUF):
    tt = NBLK - NBUF + b
    j = tt // NCH
    c0 = (tt - j * NCH) * CH
    dst = out_hbm.at[j, pl.ds(c0, CH), pl.ds(b0, BPW)]
    pltpu.make_async_copy(bufs[b], dst, sems[b]).wait()


@jax.jit
def _one_hot_sc(xt):
  mesh = plsc.VectorSubcoreMesh(core_axis_name="c", subcore_axis_name="s")
  k = pl.kernel(
      _body,
      out_type=jax.ShapeDtypeStruct((S, NUM_CLASSES, B), jnp.int32),
      mesh=mesh,
      scratch_types=[
          pltpu.VMEM((S, BPW), jnp.int32),
          pltpu.VMEM((CH, BPW), jnp.int32),
          pltpu.VMEM((CH, BPW), jnp.int32),
          pltpu.SemaphoreType.DMA,
          pltpu.SemaphoreType.DMA,
      ],
      compiler_params=pltpu.CompilerParams(
          needs_layout_passes=False,
          use_tc_tiling_on_sc=True,
      ),
  )
  return k(xt)


def kernel(x):
  xt = x.astype(jnp.int32).T
  out_t = _one_hot_sc(xt)
  return jnp.transpose(out_t, (2, 0, 1))


# speedup vs baseline: 5.8929x; 1.0137x over previous
import jax
import jax.numpy as jnp
from jax import lax
from jax.experimental import pallas as pl
from jax.experimental.pallas import tpu as pltpu
from jax.experimental.pallas import tpu_sc as plsc

NUM_CLASSES = 1000
B, S = 4096, 20
NC, NS, L = 2, 16, 16
NW = NC * NS
BPW = B // NW
GPW = BPW // L
CH = 200
NCH = NUM_CLASSES // CH
NBLK = S * NCH
NBUF = 2


def _body(xt_hbm, out_hbm, idx_v, buf0, buf1, sem0, sem1):
  wid = lax.axis_index("s") * NC + lax.axis_index("c")
  b0 = wid * BPW

  idx_cp = pltpu.async_copy(xt_hbm.at[:, pl.ds(b0, BPW)], idx_v, sem0)

  zeros = jnp.zeros((L,), jnp.int32)
  ones = jnp.ones((L,), jnp.int32)
  bufs = (buf0, buf1)
  sems = (sem0, sem1)

  def _zero_fill(buf):
    def _zero(c, _):
      for g in range(GPW):
        buf[c, pl.ds(g * L, L)] = zeros
        buf[c + CH // 2, pl.ds(g * L, L)] = zeros
      return 0

    lax.fori_loop(0, CH // 2, _zero, 0)

  iota = lax.iota(jnp.int32, L)

  def _scatter(buf, j, c0, val):
    for g in range(GPW):
      cvec = idx_v[j, pl.ds(g * L, L)]
      mask = (cvec >= c0) & (cvec < c0 + CH)
      plsc.store_scatter(buf, [cvec - c0, iota + g * L], val, mask=mask)

  def _dst(tt):
    j = tt // NCH
    c0 = (tt - j * NCH) * CH
    return j, c0, out_hbm.at[j, pl.ds(c0, CH), pl.ds(b0, BPW)]

  _zero_fill(buf0)
  idx_cp.wait()
  _scatter(buf0, 0, 0, ones)
  pltpu.async_copy(buf0, _dst(0)[2], sem0)
  _zero_fill(buf1)
  _scatter(buf1, 0, CH, ones)
  pltpu.async_copy(buf1, _dst(1)[2], sem1)

  def _step(i, _):
    for b in range(NBUF):
      tt = i * NBUF + b
      buf = bufs[b]
      sem = sems[b]
      j, c0, dst = _dst(tt)
      pltpu.make_async_copy(buf, dst, sem).wait()
      jp, c0p, _ = _dst(tt - NBUF)
      _scatter(buf, jp, c0p, zeros)
      _scatter(buf, j, c0, ones)
      pltpu.async_copy(buf, dst, sem)
    return 0

  lax.fori_loop(1, NBLK // NBUF, _step, 0)

  for b in range(NBUF):
    tt = NBLK - NBUF + b
    pltpu.make_async_copy(bufs[b], _dst(tt)[2], sems[b]).wait()


@jax.jit
def _one_hot_sc(xt):
  mesh = plsc.VectorSubcoreMesh(core_axis_name="c", subcore_axis_name="s")
  k = pl.kernel(
      _body,
      out_type=jax.ShapeDtypeStruct((S, NUM_CLASSES, B), jnp.int32),
      mesh=mesh,
      scratch_types=[
          pltpu.VMEM((S, BPW), jnp.int32),
          pltpu.VMEM((CH, BPW), jnp.int32),
          pltpu.VMEM((CH, BPW), jnp.int32),
          pltpu.SemaphoreType.DMA,
          pltpu.SemaphoreType.DMA,
      ],
      compiler_params=pltpu.CompilerParams(
          needs_layout_passes=False,
          use_tc_tiling_on_sc=True,
      ),
  )
  return k(xt)


def kernel(x):
  xt = x.astype(jnp.int32).T
  out_t = _one_hot_sc(xt)
  return jnp.transpose(out_t, (2, 0, 1))
